# baseline (speedup 1.0000x reference)
import functools

import jax
import jax.numpy as jnp
from jax import lax
from jax.experimental import pallas as pl
from jax.experimental.pallas import tpu as pltpu
from jax.experimental.pallas import tpu_sc as plsc

N = 10000
E = 320000
BLK = 128
NB = E // BLK
NBP = 2560
NBX = NBP + 8
NCORE = 2
NSUB = 16
NW = NCORE * NSUB
BPT = NBP // NW
HALF = BPT // 2
IDXR = HALF + 4
ACC = 10112
ZPT = ACC // NSUB
RPT = 624
TAIL = N - NSUB * RPT
PAD_DST = N
DEGW = 16


def _core_sub():
    return lax.axis_index("c"), lax.axis_index("s")


def _zero_acc(zeros_hbm, acc, s):
    pltpu.sync_copy(zeros_hbm, acc.at[pl.ds(s * ZPT, ZPT)])


def _dump_acc(acc, out_hbm, c, s):
    r0 = s * RPT
    pltpu.sync_copy(acc.at[pl.ds(r0, RPT)], out_hbm.at[c, pl.ds(r0, RPT)])

    @pl.when(s == NSUB - 1)
    def _():
        t0 = NSUB * RPT
        pltpu.sync_copy(acc.at[pl.ds(t0, TAIL)], out_hbm.at[c, pl.ds(t0, TAIL)])


def _make_deg_kernel():
    mesh = plsc.VectorSubcoreMesh(core_axis_name="c", subcore_axis_name="s")

    @functools.partial(
        pl.kernel,
        out_type=(
            jax.ShapeDtypeStruct((NCORE, N, DEGW), jnp.float32),
            jax.ShapeDtypeStruct((NCORE, N, DEGW), jnp.float32),
        ),
        mesh=mesh,
        scratch_types=[
            pltpu.VMEM_SHARED((ACC, DEGW), jnp.float32),
            pltpu.VMEM_SHARED((ACC, DEGW), jnp.float32),
            pltpu.VMEM((BPT, BLK), jnp.int32),
            pltpu.VMEM((BPT, BLK), jnp.int32),
            pltpu.VMEM((BLK, DEGW), jnp.float32),
            pltpu.SemaphoreType.DMA,
        ],
        compiler_params=pltpu.CompilerParams(use_tc_tiling_on_sc=False),
    )
    def deg_kernel(src_hbm, dst_hbm, ones_hbm, zeros_hbm,
                   outs_hbm, outd_hbm, acc_s, acc_d, src_v, dst_v, ones_v, sem):
        c, s = _core_sub()
        w = c * NSUB + s
        pltpu.sync_copy(src_hbm.at[pl.ds(w * BPT, BPT)], src_v)
        pltpu.sync_copy(dst_hbm.at[pl.ds(w * BPT, BPT)], dst_v)
        pltpu.sync_copy(ones_hbm, ones_v)
        _zero_acc(zeros_hbm, acc_s, s)
        _zero_acc(zeros_hbm, acc_d, s)
        plsc.subcore_barrier()

        def chunk(ch, carry):
            g0 = ch * 8
            descs = []
            for j in range(8):
                descs.append(pltpu.async_copy(
                    ones_v, acc_s.at[src_v.at[g0 + j]], sem, add=True))
                descs.append(pltpu.async_copy(
                    ones_v, acc_d.at[dst_v.at[g0 + j]], sem, add=True))
            for d in descs:
                d.wait()
            return carry

        lax.fori_loop(0, BPT // 8, chunk, 0)
        plsc.subcore_barrier()
        _dump_acc(acc_s, outs_hbm, c, s)
        _dump_acc(acc_d, outd_hbm, c, s)

    return deg_kernel


def _make_agg_kernel(D):
    mesh = plsc.VectorSubcoreMesh(core_axis_name="c", subcore_axis_name="s")

    @functools.partial(
        pl.kernel,
        out_type=jax.ShapeDtypeStruct((NCORE, N, D), jnp.float32),
        mesh=mesh,
        scratch_types=[
            pltpu.VMEM_SHARED((ACC, D), jnp.float32),
            pltpu.VMEM((BLK,), jnp.int32),
            pltpu.VMEM((BLK,), jnp.int32),
            pltpu.VMEM((8, BLK), jnp.int32),
            [pltpu.VMEM((BLK, D), jnp.float32) for _ in range(2)],
            [pltpu.SemaphoreType.DMA for _ in range(2)],
            [pltpu.SemaphoreType.DMA for _ in range(2)],
        ],
        compiler_params=pltpu.CompilerParams(use_tc_tiling_on_sc=False),
    )
    def agg_kernel(y_hbm, src_hbm, dst_hbm, pad_hbm, zeros_hbm,
                   out_hbm, acc, src_v, dst_v, pad_v, rows, gsems, ssems):
        c, s = _core_sub()
        base = (c * NSUB + s) * BPT
        pltpu.sync_copy(pad_hbm, pad_v)
        _zero_acc(zeros_hbm, acc, s)
        plsc.subcore_barrier()

        def body(g, carry):
            pltpu.sync_copy(src_hbm.at[base + g], src_v)
            pltpu.sync_copy(dst_hbm.at[base + g], dst_v)
            pltpu.async_copy(y_hbm.at[src_v], rows[0], gsems[0]).wait()
            pltpu.async_copy(rows[0], acc.at[dst_v], ssems[0], add=True).wait()
            return carry

        lax.fori_loop(0, BPT, body, 0)

        plsc.subcore_barrier()
        _dump_acc(acc, out_hbm, c, s)

    return agg_kernel


_deg_kernel = _make_deg_kernel()
_agg128 = _make_agg_kernel(128)
_agg64 = _make_agg_kernel(64)


_RB = 1000
_GRID = N // _RB


def _tc0_body(hs_ref, hd_ref, x_ref, ns_ref, nd_ref, y0_ref):
    ds = jnp.sum(hs_ref[...], axis=(0, 2)) * (1.0 / DEGW)
    dd = jnp.sum(hd_ref[...], axis=(0, 2)) * (1.0 / DEGW)
    ns = lax.rsqrt(jnp.maximum(ds, 1.0))
    nd = lax.rsqrt(jnp.maximum(dd, 1.0))
    ns_ref[...] = ns[:, None]
    nd_ref[...] = nd[:, None]
    y0_ref[...] = x_ref[...] * ns[:, None]


def _tc0(hs, hd, x):
    return pl.pallas_call(
        _tc0_body,
        grid=(_GRID,),
        in_specs=[
            pl.BlockSpec((NCORE, _RB, DEGW), lambda i: (0, i, 0)),
            pl.BlockSpec((NCORE, _RB, DEGW), lambda i: (0, i, 0)),
            pl.BlockSpec((_RB, 128), lambda i: (i, 0)),
        ],
        out_specs=[
            pl.BlockSpec((_RB, 1), lambda i: (i, 0)),
            pl.BlockSpec((_RB, 1), lambda i: (i, 0)),
            pl.BlockSpec((_RB, 128), lambda i: (i, 0)),
        ],
        out_shape=[
            jax.ShapeDtypeStruct((N, 1), jnp.float32),
            jax.ShapeDtypeStruct((N, 1), jnp.float32),
            jax.ShapeDtypeStruct((N, 128), jnp.float32),
        ],
    )(hs, hd, x)


def _tc1_body(g0_ref, ns_ref, nd_ref, w0_ref, b0_ref, y1_ref):
    g0 = (g0_ref[0] + g0_ref[1]) * nd_ref[...]
    h = jnp.dot(g0, w0_ref[...], preferred_element_type=jnp.float32)
    h = jnp.maximum(h + b0_ref[...], 0.0) * ns_ref[...]
    y1_ref[0] = h[:, :128]
    y1_ref[1] = h[:, 128:]


def _tc1(g0, ns, nd, W0, b0):
    return pl.pallas_call(
        _tc1_body,
        grid=(_GRID,),
        in_specs=[
            pl.BlockSpec((NCORE, _RB, 128), lambda i: (0, i, 0)),
            pl.BlockSpec((_RB, 1), lambda i: (i, 0)),
            pl.BlockSpec((_RB, 1), lambda i: (i, 0)),
            pl.BlockSpec((128, 256), lambda i: (0, 0)),
            pl.BlockSpec((1, 256), lambda i: (0, 0)),
        ],
        out_specs=pl.BlockSpec((NCORE, _RB, 128), lambda i: (0, i, 0)),
        out_shape=jax.ShapeDtypeStruct((NCORE, N, 128), jnp.float32),
    )(g0, ns, nd, W0, b0)


def _tc2_body(g1a_ref, g1b_ref, ns_ref, nd_ref, w1_ref, b1_ref, w2_ref, y2_ref):
    a = (g1a_ref[0] + g1a_ref[1]) * nd_ref[...]
    b = (g1b_ref[0] + g1b_ref[1]) * nd_ref[...]
    h = jnp.dot(a, w1_ref[0], preferred_element_type=jnp.float32)
    h = h + jnp.dot(b, w1_ref[1], preferred_element_type=jnp.float32)
    h = jnp.maximum(h + b1_ref[...], 0.0)
    t = jnp.dot(h, w2_ref[...], preferred_element_type=jnp.float32)
    y2_ref[...] = t * ns_ref[...]


def _tc2(g1a, g1b, ns, nd, W1, b1, W2):
    return pl.pallas_call(
        _tc2_body,
        grid=(_GRID,),
        in_specs=[
            pl.BlockSpec((NCORE, _RB, 128), lambda i: (0, i, 0)),
            pl.BlockSpec((NCORE, _RB, 128), lambda i: (0, i, 0)),
            pl.BlockSpec((_RB, 1), lambda i: (i, 0)),
            pl.BlockSpec((_RB, 1), lambda i: (i, 0)),
            pl.BlockSpec((NCORE, 128, 256), lambda i: (0, 0, 0)),
            pl.BlockSpec((1, 256), lambda i: (0, 0)),
            pl.BlockSpec((256, 64), lambda i: (0, 0)),
        ],
        out_specs=pl.BlockSpec((_RB, 64), lambda i: (i, 0)),
        out_shape=jax.ShapeDtypeStruct((N, 64), jnp.float32),
    )(g1a, g1b, ns, nd, W1, b1, W2)


def _tc3_body(g2_ref, nd_ref, b2_ref, out_ref):
    out_ref[...] = (g2_ref[0] + g2_ref[1]) * nd_ref[...] + b2_ref[...]


def _tc3(g2, nd, b2):
    return pl.pallas_call(
        _tc3_body,
        grid=(_GRID,),
        in_specs=[
            pl.BlockSpec((NCORE, _RB, 64), lambda i: (0, i, 0)),
            pl.BlockSpec((_RB, 1), lambda i: (i, 0)),
            pl.BlockSpec((1, 64), lambda i: (0, 0)),
        ],
        out_specs=pl.BlockSpec((_RB, 64), lambda i: (i, 0)),
        out_shape=jax.ShapeDtypeStruct((N, 64), jnp.float32),
    )(g2, nd, b2)


@jax.jit
def _run(x, edge_index, W0, b0, W1, b1, W2, b2):
    src = edge_index[0].astype(jnp.int32).reshape(NB, BLK)
    dst = edge_index[1].astype(jnp.int32).reshape(NB, BLK)
    padn = jnp.full((NBX - NB, BLK), PAD_DST, jnp.int32)
    pad0 = jnp.zeros((NBX - NB, BLK), jnp.int32)
    src_deg = jnp.concatenate([src, padn[:NBP - NB]])
    src_agg = jnp.concatenate([src, pad0])
    dst_pad = jnp.concatenate([dst, padn])
    ones = jnp.ones((BLK, DEGW), jnp.float32)
    pad_idx = jnp.full((8, BLK), PAD_DST, jnp.int32)
    zeros_deg = jnp.zeros((ZPT, DEGW), jnp.float32)
    zeros128 = jnp.zeros((ZPT, 128), jnp.float32)
    zeros64 = jnp.zeros((ZPT, 64), jnp.float32)

    hs, hd = _deg_kernel(src_deg, dst_pad[:NBP], ones, zeros_deg)
    ns, nd, y0 = _tc0(hs, hd, x)
    g0 = _agg128(y0, src_agg, dst_pad, pad_idx, zeros128)
    y1 = _tc1(g0, ns, nd, W0, b0.reshape(1, -1))
    g1a = _agg128(y1[0], src_agg, dst_pad, pad_idx, zeros128)
    g1b = _agg128(y1[1], src_agg, dst_pad, pad_idx, zeros128)
    y2 = _tc2(g1a, g1b, ns, nd, W1.reshape(NCORE, 128, 256),
              b1.reshape(1, -1), W2)
    g2 = _agg64(y2, src_agg, dst_pad, pad_idx, zeros64)
    return _tc3(g2, nd, b2.reshape(1, -1))


def kernel(x, edge_index, W0, b0, W1, b1, W2, b2):
    return _run(x, edge_index, W0, b0, W1, b1, W2, b2)


# speedup vs baseline: 1.1829x; 1.1829x over previous
import functools

import jax
import jax.numpy as jnp
from jax import lax
from jax.experimental import pallas as pl
from jax.experimental.pallas import tpu as pltpu
from jax.experimental.pallas import tpu_sc as plsc

N = 10000
E = 320000
BLK = 128
NB = E // BLK
NBP = 2560
NBX = NBP + 8
NCORE = 2
NSUB = 16
NW = NCORE * NSUB
BPT = NBP // NW
HALF = BPT // 2
IDXR = HALF + 4
ACC = 10112
ZPT = ACC // NSUB
RPT = 624
TAIL = N - NSUB * RPT
PAD_DST = N
DEGW = 16


def _core_sub():
    return lax.axis_index("c"), lax.axis_index("s")


def _zero_acc(zeros_hbm, acc, s):
    pltpu.sync_copy(zeros_hbm, acc.at[pl.ds(s * ZPT, ZPT)])


def _dump_acc(acc, out_hbm, c, s):
    r0 = s * RPT
    pltpu.sync_copy(acc.at[pl.ds(r0, RPT)], out_hbm.at[c, pl.ds(r0, RPT)])

    @pl.when(s == NSUB - 1)
    def _():
        t0 = NSUB * RPT
        pltpu.sync_copy(acc.at[pl.ds(t0, TAIL)], out_hbm.at[c, pl.ds(t0, TAIL)])


def _make_deg_kernel():
    mesh = plsc.VectorSubcoreMesh(core_axis_name="c", subcore_axis_name="s")

    @functools.partial(
        pl.kernel,
        out_type=(
            jax.ShapeDtypeStruct((NCORE, N, DEGW), jnp.float32),
            jax.ShapeDtypeStruct((NCORE, N, DEGW), jnp.float32),
        ),
        mesh=mesh,
        scratch_types=[
            pltpu.VMEM_SHARED((ACC, DEGW), jnp.float32),
            pltpu.VMEM_SHARED((ACC, DEGW), jnp.float32),
            pltpu.VMEM((BPT, BLK), jnp.int32),
            pltpu.VMEM((BPT, BLK), jnp.int32),
            pltpu.VMEM((BLK, DEGW), jnp.float32),
            pltpu.SemaphoreType.DMA,
        ],
        compiler_params=pltpu.CompilerParams(use_tc_tiling_on_sc=False),
    )
    def deg_kernel(src_hbm, dst_hbm, ones_hbm, zeros_hbm,
                   outs_hbm, outd_hbm, acc_s, acc_d, src_v, dst_v, ones_v, sem):
        c, s = _core_sub()
        w = c * NSUB + s
        pltpu.sync_copy(src_hbm.at[pl.ds(w * BPT, BPT)], src_v)
        pltpu.sync_copy(dst_hbm.at[pl.ds(w * BPT, BPT)], dst_v)
        pltpu.sync_copy(ones_hbm, ones_v)
        _zero_acc(zeros_hbm, acc_s, s)
        _zero_acc(zeros_hbm, acc_d, s)
        plsc.subcore_barrier()

        def chunk(ch, carry):
            g0 = ch * 8
            descs = []
            for j in range(8):
                descs.append(pltpu.async_copy(
                    ones_v, acc_s.at[src_v.at[g0 + j]], sem, add=True))
                descs.append(pltpu.async_copy(
                    ones_v, acc_d.at[dst_v.at[g0 + j]], sem, add=True))
            for d in descs:
                d.wait()
            return carry

        lax.fori_loop(0, BPT // 8, chunk, 0)
        plsc.subcore_barrier()
        _dump_acc(acc_s, outs_hbm, c, s)
        _dump_acc(acc_d, outd_hbm, c, s)

    return deg_kernel


def _make_agg_kernel(D):
    mesh = plsc.VectorSubcoreMesh(core_axis_name="c", subcore_axis_name="s")

    @functools.partial(
        pl.kernel,
        out_type=jax.ShapeDtypeStruct((NCORE, N, D), jnp.float32),
        mesh=mesh,
        scratch_types=[
            pltpu.VMEM_SHARED((ACC, D), jnp.float32),
            pltpu.VMEM((BLK,), jnp.int32),
            pltpu.VMEM((BLK,), jnp.int32),
            pltpu.VMEM((8, BLK), jnp.int32),
            [pltpu.VMEM((BLK, D), jnp.float32) for _ in range(2)],
            [pltpu.SemaphoreType.DMA for _ in range(2)],
            [pltpu.SemaphoreType.DMA for _ in range(2)],
        ],
        compiler_params=pltpu.CompilerParams(use_tc_tiling_on_sc=False),
    )
    def agg_kernel(y_hbm, src_hbm, dst_hbm, pad_hbm, zeros_hbm,
                   out_hbm, acc, src_v, dst_v, pad_v, rows, gsems, ssems):
        c, s = _core_sub()
        base = (c * NSUB + s) * BPT
        pltpu.sync_copy(pad_hbm, pad_v)
        _zero_acc(zeros_hbm, acc, s)
        plsc.subcore_barrier()

        def body(g, carry):
            pltpu.sync_copy(src_hbm.at[base + g], src_v)
            pltpu.sync_copy(dst_hbm.at[base + g], dst_v)
            pltpu.async_copy(y_hbm.at[src_v], rows[0], gsems[0]).wait()
            pltpu.async_copy(rows[0], acc.at[dst_v], ssems[0], add=True).wait()
            return carry

        lax.fori_loop(0, BPT, body, 0)

        plsc.subcore_barrier()
        _dump_acc(acc, out_hbm, c, s)

    return agg_kernel


_deg_kernel = _make_deg_kernel()
_agg128 = _make_agg_kernel(128)
_agg64 = _make_agg_kernel(64)


_RB = 1000
_GRID = N // _RB


def _tc0_body(hs_ref, hd_ref, x_ref, ns_ref, nd_ref, y0_ref):
    ds = jnp.sum(hs_ref[...], axis=(0, 2)) * (1.0 / DEGW)
    dd = jnp.sum(hd_ref[...], axis=(0, 2)) * (1.0 / DEGW)
    ns = lax.rsqrt(jnp.maximum(ds, 1.0))
    nd = lax.rsqrt(jnp.maximum(dd, 1.0))
    ns_ref[...] = ns[:, None]
    nd_ref[...] = nd[:, None]
    y0_ref[...] = x_ref[...] * ns[:, None]


def _tc0(hs, hd, x):
    return pl.pallas_call(
        _tc0_body,
        grid=(_GRID,),
        in_specs=[
            pl.BlockSpec((NCORE, _RB, DEGW), lambda i: (0, i, 0)),
            pl.BlockSpec((NCORE, _RB, DEGW), lambda i: (0, i, 0)),
            pl.BlockSpec((_RB, 128), lambda i: (i, 0)),
        ],
        out_specs=[
            pl.BlockSpec((_RB, 1), lambda i: (i, 0)),
            pl.BlockSpec((_RB, 1), lambda i: (i, 0)),
            pl.BlockSpec((_RB, 128), lambda i: (i, 0)),
        ],
        out_shape=[
            jax.ShapeDtypeStruct((N, 1), jnp.float32),
            jax.ShapeDtypeStruct((N, 1), jnp.float32),
            jax.ShapeDtypeStruct((N, 128), jnp.float32),
        ],
    )(hs, hd, x)


def _tc1_body(g0_ref, ns_ref, nd_ref, w0_ref, b0_ref, y1_ref):
    g0 = (g0_ref[0] + g0_ref[1]) * nd_ref[...]
    h = jnp.dot(g0, w0_ref[...], preferred_element_type=jnp.float32)
    h = jnp.maximum(h + b0_ref[...], 0.0) * ns_ref[...]
    y1_ref[0] = h[:, :128]
    y1_ref[1] = h[:, 128:]


def _tc1(g0, ns, nd, W0, b0):
    return pl.pallas_call(
        _tc1_body,
        grid=(_GRID,),
        in_specs=[
            pl.BlockSpec((NCORE, _RB, 128), lambda i: (0, i, 0)),
            pl.BlockSpec((_RB, 1), lambda i: (i, 0)),
            pl.BlockSpec((_RB, 1), lambda i: (i, 0)),
            pl.BlockSpec((128, 256), lambda i: (0, 0)),
            pl.BlockSpec((1, 256), lambda i: (0, 0)),
        ],
        out_specs=pl.BlockSpec((NCORE, _RB, 128), lambda i: (0, i, 0)),
        out_shape=jax.ShapeDtypeStruct((NCORE, N, 128), jnp.float32),
    )(g0, ns, nd, W0, b0)


def _tc2_body(g1a_ref, g1b_ref, ns_ref, nd_ref, w1_ref, b1_ref, w2_ref, y2_ref):
    a = (g1a_ref[0] + g1a_ref[1]) * nd_ref[...]
    b = (g1b_ref[0] + g1b_ref[1]) * nd_ref[...]
    h = jnp.dot(a, w1_ref[0], preferred_element_type=jnp.float32)
    h = h + jnp.dot(b, w1_ref[1], preferred_element_type=jnp.float32)
    h = jnp.maximum(h + b1_ref[...], 0.0)
    t = jnp.dot(h, w2_ref[...], preferred_element_type=jnp.float32)
    y2_ref[...] = t * ns_ref[...]


def _tc2(g1a, g1b, ns, nd, W1, b1, W2):
    return pl.pallas_call(
        _tc2_body,
        grid=(_GRID,),
        in_specs=[
            pl.BlockSpec((NCORE, _RB, 128), lambda i: (0, i, 0)),
            pl.BlockSpec((NCORE, _RB, 128), lambda i: (0, i, 0)),
            pl.BlockSpec((_RB, 1), lambda i: (i, 0)),
            pl.BlockSpec((_RB, 1), lambda i: (i, 0)),
            pl.BlockSpec((NCORE, 128, 256), lambda i: (0, 0, 0)),
            pl.BlockSpec((1, 256), lambda i: (0, 0)),
            pl.BlockSpec((256, 64), lambda i: (0, 0)),
        ],
        out_specs=pl.BlockSpec((_RB, 64), lambda i: (i, 0)),
        out_shape=jax.ShapeDtypeStruct((N, 64), jnp.float32),
    )(g1a, g1b, ns, nd, W1, b1, W2)


def _tc3_body(g2_ref, nd_ref, b2_ref, out_ref):
    out_ref[...] = (g2_ref[0] + g2_ref[1]) * nd_ref[...] + b2_ref[...]


def _tc3(g2, nd, b2):
    return pl.pallas_call(
        _tc3_body,
        grid=(_GRID,),
        in_specs=[
            pl.BlockSpec((NCORE, _RB, 64), lambda i: (0, i, 0)),
            pl.BlockSpec((_RB, 1), lambda i: (i, 0)),
            pl.BlockSpec((1, 64), lambda i: (0, 0)),
        ],
        out_specs=pl.BlockSpec((_RB, 64), lambda i: (i, 0)),
        out_shape=jax.ShapeDtypeStruct((N, 64), jnp.float32),
    )(g2, nd, b2)


@jax.jit
def _run(x, edge_index, W0, b0, W1, b1, W2, b2):
    src = edge_index[0].astype(jnp.int32).reshape(NB, BLK)
    dst = edge_index[1].astype(jnp.int32).reshape(NB, BLK)
    spread = PAD_DST + jnp.arange((NBX - NB) * BLK, dtype=jnp.int32) % (ACC - N)
    padn = spread.reshape(NBX - NB, BLK)
    pad0 = jnp.zeros((NBX - NB, BLK), jnp.int32)
    src_deg = jnp.concatenate([src, padn[:NBP - NB]])
    src_agg = jnp.concatenate([src, pad0])
    dst_pad = jnp.concatenate([dst, padn])
    ones = jnp.ones((BLK, DEGW), jnp.float32)
    pad_idx = PAD_DST + jnp.arange(8 * BLK, dtype=jnp.int32) % (ACC - N)
    pad_idx = pad_idx.reshape(8, BLK)
    zeros_deg = jnp.zeros((ZPT, DEGW), jnp.float32)
    zeros128 = jnp.zeros((ZPT, 128), jnp.float32)
    zeros64 = jnp.zeros((ZPT, 64), jnp.float32)

    hs, hd = _deg_kernel(src_deg, dst_pad[:NBP], ones, zeros_deg)
    ns, nd, y0 = _tc0(hs, hd, x)
    g0 = _agg128(y0, src_agg, dst_pad, pad_idx, zeros128)
    y1 = _tc1(g0, ns, nd, W0, b0.reshape(1, -1))
    g1a = _agg128(y1[0], src_agg, dst_pad, pad_idx, zeros128)
    g1b = _agg128(y1[1], src_agg, dst_pad, pad_idx, zeros128)
    y2 = _tc2(g1a, g1b, ns, nd, W1.reshape(NCORE, 128, 256),
              b1.reshape(1, -1), W2)
    g2 = _agg64(y2, src_agg, dst_pad, pad_idx, zeros64)
    return _tc3(g2, nd, b2.reshape(1, -1))


def kernel(x, edge_index, W0, b0, W1, b1, W2, b2):
    return _run(x, edge_index, W0, b0, W1, b1, W2, b2)


# speedup vs baseline: 2.3519x; 1.9882x over previous
import functools

import numpy as np

import jax
import jax.numpy as jnp
from jax import lax
from jax.experimental import pallas as pl
from jax.experimental.pallas import tpu as pltpu
from jax.experimental.pallas import tpu_sc as plsc

N = 10000
E = 320000
BLK = 128
NB = E // BLK
NCORE = 2
NSUB = 16
NW = NCORE * NSUB
BPT = 79
NBP = NW * BPT
NBX = NBP + 8
HALF = 40
IDXR = HALF + 4
ACC = 10112
ZPT = ACC // NSUB
RPT = 624
TAIL = N - NSUB * RPT
PAD_DST = N
DEGW = 16


def _core_sub():
    return lax.axis_index("c"), lax.axis_index("s")


def _zero_acc(zeros_hbm, acc, s):
    pltpu.sync_copy(zeros_hbm, acc.at[pl.ds(s * ZPT, ZPT)])


def _dump_acc(acc, out_hbm, c, s):
    r0 = s * RPT
    pltpu.sync_copy(acc.at[pl.ds(r0, RPT)], out_hbm.at[c, pl.ds(r0, RPT)])

    @pl.when(s == NSUB - 1)
    def _():
        t0 = NSUB * RPT
        pltpu.sync_copy(acc.at[pl.ds(t0, TAIL)], out_hbm.at[c, pl.ds(t0, TAIL)])


def _make_deg_kernel():
    mesh = plsc.VectorSubcoreMesh(core_axis_name="c", subcore_axis_name="s")

    @functools.partial(
        pl.kernel,
        out_type=(
            jax.ShapeDtypeStruct((NCORE, N, DEGW), jnp.float32),
            jax.ShapeDtypeStruct((NCORE, N, DEGW), jnp.float32),
        ),
        mesh=mesh,
        scratch_types=[
            pltpu.VMEM_SHARED((ACC, DEGW), jnp.float32),
            pltpu.VMEM_SHARED((ACC, DEGW), jnp.float32),
            pltpu.VMEM((BPT, BLK), jnp.int32),
            pltpu.VMEM((BPT, BLK), jnp.int32),
            pltpu.VMEM((BLK, DEGW), jnp.float32),
            pltpu.SemaphoreType.DMA,
        ],
        compiler_params=pltpu.CompilerParams(use_tc_tiling_on_sc=False),
    )
    def deg_kernel(src_hbm, dst_hbm, ones_hbm, zeros_hbm,
                   outs_hbm, outd_hbm, acc_s, acc_d, src_v, dst_v, ones_v, sem):
        c, s = _core_sub()
        w = c * NSUB + s
        pltpu.sync_copy(src_hbm.at[pl.ds(w * BPT, BPT)], src_v)
        pltpu.sync_copy(dst_hbm.at[pl.ds(w * BPT, BPT)], dst_v)
        pltpu.sync_copy(ones_hbm, ones_v)
        _zero_acc(zeros_hbm, acc_s, s)
        _zero_acc(zeros_hbm, acc_d, s)
        plsc.subcore_barrier()

        def run_blocks(g0, count):
            descs = []
            for j in range(count):
                descs.append(pltpu.async_copy(
                    ones_v, acc_s.at[src_v.at[g0 + j]], sem, add=True))
                descs.append(pltpu.async_copy(
                    ones_v, acc_d.at[dst_v.at[g0 + j]], sem, add=True))
            for d in descs:
                d.wait()

        def chunk(ch, carry):
            run_blocks(ch * 8, 8)
            return carry

        lax.fori_loop(0, BPT // 8, chunk, 0)
        run_blocks((BPT // 8) * 8, BPT % 8)
        plsc.subcore_barrier()
        _dump_acc(acc_s, outs_hbm, c, s)
        _dump_acc(acc_d, outd_hbm, c, s)

    return deg_kernel


def _make_agg_kernel(D):
    mesh = plsc.VectorSubcoreMesh(core_axis_name="c", subcore_axis_name="s")

    @functools.partial(
        pl.kernel,
        out_type=jax.ShapeDtypeStruct((NCORE, N, D), jnp.float32),
        mesh=mesh,
        scratch_types=[
            pltpu.VMEM_SHARED((ACC, D), jnp.float32),
            pltpu.VMEM((IDXR, BLK), jnp.int32),
            pltpu.VMEM((IDXR, BLK), jnp.int32),
            pltpu.VMEM((8, BLK), jnp.int32),
            [pltpu.VMEM((BLK, D), jnp.float32) for _ in range(2)],
            [pltpu.SemaphoreType.DMA for _ in range(2)],
            [pltpu.SemaphoreType.DMA for _ in range(2)],
        ],
        compiler_params=pltpu.CompilerParams(use_tc_tiling_on_sc=False),
    )
    def agg_kernel(y_hbm, src_hbm, dst_hbm, pad_hbm, zeros_hbm,
                   out_hbm, acc, src_v, dst_v, pad_v, rows, gsems, ssems):
        c, s = _core_sub()
        base = (c * NSUB + s) * BPT
        pltpu.sync_copy(pad_hbm, pad_v)
        _zero_acc(zeros_hbm, acc, s)
        plsc.subcore_barrier()

        def gather(g, b):
            return pltpu.async_copy(y_hbm.at[src_v.at[g]], rows[b], gsems[b])

        def scatter(g, b):
            return pltpu.async_copy(rows[b], acc.at[dst_v.at[g]], ssems[b],
                                    add=True)

        def gwait(b):
            pltpu.make_async_copy(y_hbm.at[src_v.at[0]], rows[b],
                                  gsems[b]).wait()

        def swait(b):
            pltpu.make_async_copy(rows[b], acc.at[pad_v.at[0]],
                                  ssems[b]).wait()

        for h, nb in ((0, HALF), (1, BPT - HALF)):
            off = base + h * HALF
            pltpu.sync_copy(src_hbm.at[pl.ds(off, IDXR)], src_v)
            pltpu.sync_copy(dst_hbm.at[pl.ds(off, IDXR)], dst_v)
            pltpu.async_copy(rows[1], acc.at[pad_v.at[0]], ssems[1], add=True)
            gather(0, 0)

            def pair(i, carry):
                g = 2 * i
                swait(1)
                gather(g + 1, 1)
                gwait(0)
                scatter(g, 0)
                swait(0)
                gather(g + 2, 0)
                gwait(1)
                scatter(g + 1, 1)
                return carry

            lax.fori_loop(0, nb // 2, pair, 0)
            if nb % 2:
                g = nb - 1
                swait(1)
                gather(g + 1, 1)
                gwait(0)
                scatter(g, 0)
                swait(0)
                gwait(1)
            else:
                swait(1)
                gwait(0)

        plsc.subcore_barrier()
        _dump_acc(acc, out_hbm, c, s)

    return agg_kernel


_deg_kernel = _make_deg_kernel()
_agg128 = _make_agg_kernel(128)
_agg64 = _make_agg_kernel(64)


_RB = 1000
_GRID = N // _RB


def _tc0_body(hs_ref, hd_ref, x_ref, ns_ref, nd_ref, y0_ref):
    ds = jnp.sum(hs_ref[...], axis=(0, 2)) * (1.0 / DEGW)
    dd = jnp.sum(hd_ref[...], axis=(0, 2)) * (1.0 / DEGW)
    ns = lax.rsqrt(jnp.maximum(ds, 1.0))
    nd = lax.rsqrt(jnp.maximum(dd, 1.0))
    ns_ref[...] = ns[:, None]
    nd_ref[...] = nd[:, None]
    y0_ref[...] = x_ref[...] * ns[:, None]


def _tc0(hs, hd, x):
    return pl.pallas_call(
        _tc0_body,
        grid=(_GRID,),
        in_specs=[
            pl.BlockSpec((NCORE, _RB, DEGW), lambda i: (0, i, 0)),
            pl.BlockSpec((NCORE, _RB, DEGW), lambda i: (0, i, 0)),
            pl.BlockSpec((_RB, 128), lambda i: (i, 0)),
        ],
        out_specs=[
            pl.BlockSpec((_RB, 1), lambda i: (i, 0)),
            pl.BlockSpec((_RB, 1), lambda i: (i, 0)),
            pl.BlockSpec((_RB, 128), lambda i: (i, 0)),
        ],
        out_shape=[
            jax.ShapeDtypeStruct((N, 1), jnp.float32),
            jax.ShapeDtypeStruct((N, 1), jnp.float32),
            jax.ShapeDtypeStruct((N, 128), jnp.float32),
        ],
    )(hs, hd, x)


def _tc1_body(g0_ref, ns_ref, nd_ref, w0_ref, b0_ref, y1_ref):
    g0 = (g0_ref[0] + g0_ref[1]) * nd_ref[...]
    h = jnp.dot(g0, w0_ref[...], preferred_element_type=jnp.float32)
    h = jnp.maximum(h + b0_ref[...], 0.0) * ns_ref[...]
    y1_ref[0] = h[:, :128]
    y1_ref[1] = h[:, 128:]


def _tc1(g0, ns, nd, W0, b0):
    return pl.pallas_call(
        _tc1_body,
        grid=(_GRID,),
        in_specs=[
            pl.BlockSpec((NCORE, _RB, 128), lambda i: (0, i, 0)),
            pl.BlockSpec((_RB, 1), lambda i: (i, 0)),
            pl.BlockSpec((_RB, 1), lambda i: (i, 0)),
            pl.BlockSpec((128, 256), lambda i: (0, 0)),
            pl.BlockSpec((1, 256), lambda i: (0, 0)),
        ],
        out_specs=pl.BlockSpec((NCORE, _RB, 128), lambda i: (0, i, 0)),
        out_shape=jax.ShapeDtypeStruct((NCORE, N, 128), jnp.float32),
    )(g0, ns, nd, W0, b0)


def _tc2_body(g1a_ref, g1b_ref, ns_ref, nd_ref, w1_ref, b1_ref, w2_ref, y2_ref):
    a = (g1a_ref[0] + g1a_ref[1]) * nd_ref[...]
    b = (g1b_ref[0] + g1b_ref[1]) * nd_ref[...]
    h = jnp.dot(a, w1_ref[0], preferred_element_type=jnp.float32)
    h = h + jnp.dot(b, w1_ref[1], preferred_element_type=jnp.float32)
    h = jnp.maximum(h + b1_ref[...], 0.0)
    t = jnp.dot(h, w2_ref[...], preferred_element_type=jnp.float32)
    y2_ref[...] = t * ns_ref[...]


def _tc2(g1a, g1b, ns, nd, W1, b1, W2):
    return pl.pallas_call(
        _tc2_body,
        grid=(_GRID,),
        in_specs=[
            pl.BlockSpec((NCORE, _RB, 128), lambda i: (0, i, 0)),
            pl.BlockSpec((NCORE, _RB, 128), lambda i: (0, i, 0)),
            pl.BlockSpec((_RB, 1), lambda i: (i, 0)),
            pl.BlockSpec((_RB, 1), lambda i: (i, 0)),
            pl.BlockSpec((NCORE, 128, 256), lambda i: (0, 0, 0)),
            pl.BlockSpec((1, 256), lambda i: (0, 0)),
            pl.BlockSpec((256, 64), lambda i: (0, 0)),
        ],
        out_specs=pl.BlockSpec((_RB, 64), lambda i: (i, 0)),
        out_shape=jax.ShapeDtypeStruct((N, 64), jnp.float32),
    )(g1a, g1b, ns, nd, W1, b1, W2)


def _tc3_body(g2_ref, nd_ref, b2_ref, out_ref):
    out_ref[...] = (g2_ref[0] + g2_ref[1]) * nd_ref[...] + b2_ref[...]


def _tc3(g2, nd, b2):
    return pl.pallas_call(
        _tc3_body,
        grid=(_GRID,),
        in_specs=[
            pl.BlockSpec((NCORE, _RB, 64), lambda i: (0, i, 0)),
            pl.BlockSpec((_RB, 1), lambda i: (i, 0)),
            pl.BlockSpec((1, 64), lambda i: (0, 0)),
        ],
        out_specs=pl.BlockSpec((_RB, 64), lambda i: (i, 0)),
        out_shape=jax.ShapeDtypeStruct((N, 64), jnp.float32),
    )(g2, nd, b2)


def _block_perm():
    perm = np.empty(NBP, dtype=np.int32)
    ridx = 0
    for w in range(NW):
        npad = 1 if w < NBP - NB else 0
        nreal = BPT - npad
        perm[w * BPT:w * BPT + nreal] = np.arange(ridx, ridx + nreal)
        if npad:
            perm[w * BPT + BPT - 1] = NB + w
        ridx += nreal
    assert ridx == NB
    return jnp.asarray(perm)


_PERM = _block_perm()


@jax.jit
def _run(x, edge_index, W0, b0, W1, b1, W2, b2):
    src = edge_index[0].astype(jnp.int32).reshape(NB, BLK)
    dst = edge_index[1].astype(jnp.int32).reshape(NB, BLK)
    spread = PAD_DST + jnp.arange((NBX - NB) * BLK, dtype=jnp.int32) % (ACC - N)
    padn = spread.reshape(NBX - NB, BLK)
    pad0 = jnp.zeros((NBX - NB, BLK), jnp.int32)
    src_deg = jnp.take(jnp.concatenate([src, padn[:NBP - NB]]), _PERM, axis=0)
    src_agg = jnp.take(jnp.concatenate([src, pad0[:NBP - NB]]), _PERM, axis=0)
    src_agg = jnp.concatenate([src_agg, pad0[:NBX - NBP]])
    dst_pad = jnp.take(jnp.concatenate([dst, padn[:NBP - NB]]), _PERM, axis=0)
    dst_pad = jnp.concatenate([dst_pad, padn[NBP - NB:]])
    ones = jnp.ones((BLK, DEGW), jnp.float32)
    pad_idx = PAD_DST + jnp.arange(8 * BLK, dtype=jnp.int32) % (ACC - N)
    pad_idx = pad_idx.reshape(8, BLK)
    zeros_deg = jnp.zeros((ZPT, DEGW), jnp.float32)
    zeros128 = jnp.zeros((ZPT, 128), jnp.float32)
    zeros64 = jnp.zeros((ZPT, 64), jnp.float32)

    hs, hd = _deg_kernel(src_deg[:NBP], dst_pad[:NBP], ones, zeros_deg)
    ns, nd, y0 = _tc0(hs, hd, x)
    g0 = _agg128(y0, src_agg, dst_pad, pad_idx, zeros128)
    y1 = _tc1(g0, ns, nd, W0, b0.reshape(1, -1))
    g1a = _agg128(y1[0], src_agg, dst_pad, pad_idx, zeros128)
    g1b = _agg128(y1[1], src_agg, dst_pad, pad_idx, zeros128)
    y2 = _tc2(g1a, g1b, ns, nd, W1.reshape(NCORE, 128, 256),
              b1.reshape(1, -1), W2)
    g2 = _agg64(y2, src_agg, dst_pad, pad_idx, zeros64)
    return _tc3(g2, nd, b2.reshape(1, -1))


def kernel(x, edge_index, W0, b0, W1, b1, W2, b2):
    return _run(x, edge_index, W0, b0, W1, b1, W2, b2)


# speedup vs baseline: 2.3876x; 1.0152x over previous
import functools

import numpy as np

import jax
import jax.numpy as jnp
from jax import lax
from jax.experimental import pallas as pl
from jax.experimental.pallas import tpu as pltpu
from jax.experimental.pallas import tpu_sc as plsc

N = 10000
E = 320000
BLK = 128
NB = E // BLK
NCORE = 2
NSUB = 16
NW = NCORE * NSUB
BPT = 79
NBP = NW * BPT
NBX = NBP + 8
SUB = 64
SPT = 2 * BPT
SHALF = 80
IDXR = SHALF + 4
NBUF = 4
ACC = 10112
ZPT = ACC // NSUB
RPT = 624
TAIL = N - NSUB * RPT
PAD_DST = N
DEGW = 16


def _core_sub():
    return lax.axis_index("c"), lax.axis_index("s")


def _zero_acc(zeros_hbm, acc, s):
    pltpu.sync_copy(zeros_hbm, acc.at[pl.ds(s * ZPT, ZPT)])


def _dump_acc(acc, out_hbm, c, s):
    r0 = s * RPT
    pltpu.sync_copy(acc.at[pl.ds(r0, RPT)], out_hbm.at[c, pl.ds(r0, RPT)])

    @pl.when(s == NSUB - 1)
    def _():
        t0 = NSUB * RPT
        pltpu.sync_copy(acc.at[pl.ds(t0, TAIL)], out_hbm.at[c, pl.ds(t0, TAIL)])


def _make_deg_kernel():
    mesh = plsc.VectorSubcoreMesh(core_axis_name="c", subcore_axis_name="s")

    @functools.partial(
        pl.kernel,
        out_type=(
            jax.ShapeDtypeStruct((NCORE, N, DEGW), jnp.float32),
            jax.ShapeDtypeStruct((NCORE, N, DEGW), jnp.float32),
        ),
        mesh=mesh,
        scratch_types=[
            pltpu.VMEM_SHARED((ACC, DEGW), jnp.float32),
            pltpu.VMEM_SHARED((ACC, DEGW), jnp.float32),
            pltpu.VMEM((BPT, BLK), jnp.int32),
            pltpu.VMEM((BPT, BLK), jnp.int32),
            pltpu.VMEM((BLK, DEGW), jnp.float32),
            pltpu.SemaphoreType.DMA,
        ],
        compiler_params=pltpu.CompilerParams(use_tc_tiling_on_sc=False),
    )
    def deg_kernel(src_hbm, dst_hbm, ones_hbm, zeros_hbm,
                   outs_hbm, outd_hbm, acc_s, acc_d, src_v, dst_v, ones_v, sem):
        c, s = _core_sub()
        w = c * NSUB + s
        pltpu.sync_copy(src_hbm.at[pl.ds(w * BPT, BPT)], src_v)
        pltpu.sync_copy(dst_hbm.at[pl.ds(w * BPT, BPT)], dst_v)
        pltpu.sync_copy(ones_hbm, ones_v)
        _zero_acc(zeros_hbm, acc_s, s)
        _zero_acc(zeros_hbm, acc_d, s)
        plsc.subcore_barrier()

        def run_blocks(g0, count):
            descs = []
            for j in range(count):
                descs.append(pltpu.async_copy(
                    ones_v, acc_s.at[src_v.at[g0 + j]], sem, add=True))
                descs.append(pltpu.async_copy(
                    ones_v, acc_d.at[dst_v.at[g0 + j]], sem, add=True))
            for d in descs:
                d.wait()

        def chunk(ch, carry):
            run_blocks(ch * 8, 8)
            return carry

        lax.fori_loop(0, BPT // 8, chunk, 0)
        run_blocks((BPT // 8) * 8, BPT % 8)
        plsc.subcore_barrier()
        _dump_acc(acc_s, outs_hbm, c, s)
        _dump_acc(acc_d, outd_hbm, c, s)

    return deg_kernel


def _make_agg_kernel(D):
    mesh = plsc.VectorSubcoreMesh(core_axis_name="c", subcore_axis_name="s")

    @functools.partial(
        pl.kernel,
        out_type=jax.ShapeDtypeStruct((NCORE, N, D), jnp.float32),
        mesh=mesh,
        scratch_types=[
            pltpu.VMEM_SHARED((ACC, D), jnp.float32),
            pltpu.VMEM((IDXR, SUB), jnp.int32),
            pltpu.VMEM((IDXR, SUB), jnp.int32),
            pltpu.VMEM((8, SUB), jnp.int32),
            [pltpu.VMEM((SUB, D), jnp.float32) for _ in range(NBUF)],
            [pltpu.SemaphoreType.DMA for _ in range(NBUF)],
            [pltpu.SemaphoreType.DMA for _ in range(NBUF)],
        ],
        compiler_params=pltpu.CompilerParams(use_tc_tiling_on_sc=False),
    )
    def agg_kernel(y_hbm, src_hbm, dst_hbm, pad_hbm, zeros_hbm,
                   out_hbm, acc, src_v, dst_v, pad_v, rows, gsems, ssems):
        c, s = _core_sub()
        base = (c * NSUB + s) * SPT
        pltpu.sync_copy(pad_hbm, pad_v)
        _zero_acc(zeros_hbm, acc, s)
        plsc.subcore_barrier()

        def gather(g, b):
            return pltpu.async_copy(y_hbm.at[src_v.at[g]], rows[b], gsems[b])

        def scatter(g, b):
            return pltpu.async_copy(rows[b], acc.at[dst_v.at[g]], ssems[b],
                                    add=True)

        def gwait(b):
            pltpu.make_async_copy(y_hbm.at[src_v.at[0]], rows[b],
                                  gsems[b]).wait()

        def swait(b):
            pltpu.make_async_copy(rows[b], acc.at[pad_v.at[0]],
                                  ssems[b]).wait()

        def step(g, j):
            b, b2 = j % NBUF, (j + 2) % NBUF
            swait(b2)
            gather(g + 2, b2)
            gwait(b)
            scatter(g, b)

        for h, nb in ((0, SHALF), (1, SPT - SHALF)):
            off = base + h * SHALF
            pltpu.sync_copy(src_hbm.at[pl.ds(off, IDXR)], src_v)
            pltpu.sync_copy(dst_hbm.at[pl.ds(off, IDXR)], dst_v)
            pltpu.async_copy(rows[2], acc.at[pad_v.at[0]], ssems[2], add=True)
            pltpu.async_copy(rows[3], acc.at[pad_v.at[0]], ssems[3], add=True)
            gather(0, 0)
            gather(1, 1)

            def quad(i, carry):
                g0 = NBUF * i
                for j in range(NBUF):
                    step(g0 + j, j)
                return carry

            lax.fori_loop(0, nb // NBUF, quad, 0)
            t0 = nb - nb % NBUF
            for j in range(nb % NBUF):
                step(t0 + j, j)
            swait((nb - 2) % NBUF)
            swait((nb - 1) % NBUF)
            gwait(nb % NBUF)
            gwait((nb + 1) % NBUF)

        plsc.subcore_barrier()
        _dump_acc(acc, out_hbm, c, s)

    return agg_kernel


_deg_kernel = _make_deg_kernel()
_agg128 = _make_agg_kernel(128)
_agg64 = _make_agg_kernel(64)


_RB = 1000
_GRID = N // _RB


def _tc0_body(hs_ref, hd_ref, x_ref, ns_ref, nd_ref, y0_ref):
    ds = jnp.sum(hs_ref[...], axis=(0, 2)) * (1.0 / DEGW)
    dd = jnp.sum(hd_ref[...], axis=(0, 2)) * (1.0 / DEGW)
    ns = lax.rsqrt(jnp.maximum(ds, 1.0))
    nd = lax.rsqrt(jnp.maximum(dd, 1.0))
    ns_ref[...] = ns[:, None]
    nd_ref[...] = nd[:, None]
    y0_ref[...] = x_ref[...] * ns[:, None]


def _tc0(hs, hd, x):
    return pl.pallas_call(
        _tc0_body,
        grid=(_GRID,),
        in_specs=[
            pl.BlockSpec((NCORE, _RB, DEGW), lambda i: (0, i, 0)),
            pl.BlockSpec((NCORE, _RB, DEGW), lambda i: (0, i, 0)),
            pl.BlockSpec((_RB, 128), lambda i: (i, 0)),
        ],
        out_specs=[
            pl.BlockSpec((_RB, 1), lambda i: (i, 0)),
            pl.BlockSpec((_RB, 1), lambda i: (i, 0)),
            pl.BlockSpec((_RB, 128), lambda i: (i, 0)),
        ],
        out_shape=[
            jax.ShapeDtypeStruct((N, 1), jnp.float32),
            jax.ShapeDtypeStruct((N, 1), jnp.float32),
            jax.ShapeDtypeStruct((N, 128), jnp.float32),
        ],
    )(hs, hd, x)


def _tc1_body(g0_ref, ns_ref, nd_ref, w0_ref, b0_ref, y1_ref):
    g0 = (g0_ref[0] + g0_ref[1]) * nd_ref[...]
    h = jnp.dot(g0, w0_ref[...], preferred_element_type=jnp.float32)
    h = jnp.maximum(h + b0_ref[...], 0.0) * ns_ref[...]
    y1_ref[0] = h[:, :128]
    y1_ref[1] = h[:, 128:]


def _tc1(g0, ns, nd, W0, b0):
    return pl.pallas_call(
        _tc1_body,
        grid=(_GRID,),
        in_specs=[
            pl.BlockSpec((NCORE, _RB, 128), lambda i: (0, i, 0)),
            pl.BlockSpec((_RB, 1), lambda i: (i, 0)),
            pl.BlockSpec((_RB, 1), lambda i: (i, 0)),
            pl.BlockSpec((128, 256), lambda i: (0, 0)),
            pl.BlockSpec((1, 256), lambda i: (0, 0)),
        ],
        out_specs=pl.BlockSpec((NCORE, _RB, 128), lambda i: (0, i, 0)),
        out_shape=jax.ShapeDtypeStruct((NCORE, N, 128), jnp.float32),
    )(g0, ns, nd, W0, b0)


def _tc2_body(g1a_ref, g1b_ref, ns_ref, nd_ref, w1_ref, b1_ref, w2_ref, y2_ref):
    a = (g1a_ref[0] + g1a_ref[1]) * nd_ref[...]
    b = (g1b_ref[0] + g1b_ref[1]) * nd_ref[...]
    h = jnp.dot(a, w1_ref[0], preferred_element_type=jnp.float32)
    h = h + jnp.dot(b, w1_ref[1], preferred_element_type=jnp.float32)
    h = jnp.maximum(h + b1_ref[...], 0.0)
    t = jnp.dot(h, w2_ref[...], preferred_element_type=jnp.float32)
    y2_ref[...] = t * ns_ref[...]


def _tc2(g1a, g1b, ns, nd, W1, b1, W2):
    return pl.pallas_call(
        _tc2_body,
        grid=(_GRID,),
        in_specs=[
            pl.BlockSpec((NCORE, _RB, 128), lambda i: (0, i, 0)),
            pl.BlockSpec((NCORE, _RB, 128), lambda i: (0, i, 0)),
            pl.BlockSpec((_RB, 1), lambda i: (i, 0)),
            pl.BlockSpec((_RB, 1), lambda i: (i, 0)),
            pl.BlockSpec((NCORE, 128, 256), lambda i: (0, 0, 0)),
            pl.BlockSpec((1, 256), lambda i: (0, 0)),
            pl.BlockSpec((256, 64), lambda i: (0, 0)),
        ],
        out_specs=pl.BlockSpec((_RB, 64), lambda i: (i, 0)),
        out_shape=jax.ShapeDtypeStruct((N, 64), jnp.float32),
    )(g1a, g1b, ns, nd, W1, b1, W2)


def _tc3_body(g2_ref, nd_ref, b2_ref, out_ref):
    out_ref[...] = (g2_ref[0] + g2_ref[1]) * nd_ref[...] + b2_ref[...]


def _tc3(g2, nd, b2):
    return pl.pallas_call(
        _tc3_body,
        grid=(_GRID,),
        in_specs=[
            pl.BlockSpec((NCORE, _RB, 64), lambda i: (0, i, 0)),
            pl.BlockSpec((_RB, 1), lambda i: (i, 0)),
            pl.BlockSpec((1, 64), lambda i: (0, 0)),
        ],
        out_specs=pl.BlockSpec((_RB, 64), lambda i: (i, 0)),
        out_shape=jax.ShapeDtypeStruct((N, 64), jnp.float32),
    )(g2, nd, b2)


def _block_perm():
    perm = np.empty(NBP, dtype=np.int32)
    ridx = 0
    for w in range(NW):
        npad = 1 if w < NBP - NB else 0
        nreal = BPT - npad
        perm[w * BPT:w * BPT + nreal] = np.arange(ridx, ridx + nreal)
        if npad:
            perm[w * BPT + BPT - 1] = NB + w
        ridx += nreal
    assert ridx == NB
    return jnp.asarray(perm)


_PERM = _block_perm()


@jax.jit
def _run(x, edge_index, W0, b0, W1, b1, W2, b2):
    src = edge_index[0].astype(jnp.int32).reshape(NB, BLK)
    dst = edge_index[1].astype(jnp.int32).reshape(NB, BLK)
    spread = PAD_DST + jnp.arange((NBX - NB) * BLK, dtype=jnp.int32) % (ACC - N)
    padn = spread.reshape(NBX - NB, BLK)
    pad0 = jnp.zeros((NBX - NB, BLK), jnp.int32)
    src_deg = jnp.take(jnp.concatenate([src, padn[:NBP - NB]]), _PERM, axis=0)
    src_agg = jnp.take(jnp.concatenate([src, pad0[:NBP - NB]]), _PERM, axis=0)
    src_agg = jnp.concatenate([src_agg, pad0[:NBX - NBP]])
    dst_pad = jnp.take(jnp.concatenate([dst, padn[:NBP - NB]]), _PERM, axis=0)
    dst_pad = jnp.concatenate([dst_pad, padn[NBP - NB:]])
    src_sub = src_agg.reshape(2 * NBX, SUB)
    dst_sub = dst_pad.reshape(2 * NBX, SUB)
    ones = jnp.ones((BLK, DEGW), jnp.float32)
    pad_idx = PAD_DST + jnp.arange(8 * SUB, dtype=jnp.int32) % (ACC - N)
    pad_idx = pad_idx.reshape(8, SUB)
    zeros_deg = jnp.zeros((ZPT, DEGW), jnp.float32)
    zeros128 = jnp.zeros((ZPT, 128), jnp.float32)
    zeros64 = jnp.zeros((ZPT, 64), jnp.float32)

    hs, hd = _deg_kernel(src_deg[:NBP], dst_pad[:NBP], ones, zeros_deg)
    ns, nd, y0 = _tc0(hs, hd, x)
    g0 = _agg128(y0, src_sub, dst_sub, pad_idx, zeros128)
    y1 = _tc1(g0, ns, nd, W0, b0.reshape(1, -1))
    g1a = _agg128(y1[0], src_sub, dst_sub, pad_idx, zeros128)
    g1b = _agg128(y1[1], src_sub, dst_sub, pad_idx, zeros128)
    y2 = _tc2(g1a, g1b, ns, nd, W1.reshape(NCORE, 128, 256),
              b1.reshape(1, -1), W2)
    g2 = _agg64(y2, src_sub, dst_sub, pad_idx, zeros64)
    return _tc3(g2, nd, b2.reshape(1, -1))


def kernel(x, edge_index, W0, b0, W1, b1, W2, b2):
    return _run(x, edge_index, W0, b0, W1, b1, W2, b2)
